# baseline (speedup 1.0000x reference)
import jax
import jax.numpy as jnp
from jax import lax
from jax.experimental import pallas as pl
from jax.experimental.pallas import tpu as pltpu
from jax.experimental.pallas import tpu_sc as plsc

VOCAB = 1000000
EMB_DIM = 64
BATCH = 16384

_NC = 2
_NS = 16
_LANES = 16
_NW = _NC * _NS
_BPW = BATCH // _NW
_CHUNK = 64
_NCHUNK = _BPW // _CHUNK
_GROUPS = VOCAB // 16


def _emb_mul_kernel(idx_hbm, repr_hbm, table_hbm, out_hbm,
                    idx_v, gid_v, sub_v, half_v, tiles_v, rep_v, ob_v,
                    gsem, rsem):
    wid = lax.axis_index("s") * _NC + lax.axis_index("c")
    base = wid * _BPW

    pltpu.sync_copy(idx_hbm.at[pl.ds(wid * _NCHUNK, _NCHUNK)], idx_v)

    def split_body(k, carry):
        j = k // (_CHUNK // _LANES)
        t = k % (_CHUNK // _LANES)
        sl = pl.ds(t * _LANES, _LANES)
        v = idx_v[j, sl]
        gid_v[j, sl] = lax.shift_right_logical(v, 4)
        sub_v[j, sl] = lax.bitwise_and(lax.shift_right_logical(v, 1), 7)
        half_v[j, sl] = lax.bitwise_and(v, 1) * EMB_DIM
        return carry

    lax.fori_loop(0, _NCHUNK * (_CHUNK // _LANES), split_body, 0, unroll=4)

    for j in range(_NCHUNK):
        rbase = base + j * _CHUNK
        gcp = pltpu.async_copy(table_hbm.at[gid_v.at[j]], tiles_v, gsem)
        rcp = pltpu.async_copy(repr_hbm.at[pl.ds(rbase, _CHUNK)], rep_v, rsem)
        gcp.wait()
        rcp.wait()

        def mul_body(k, carry):
            sv = sub_v[j, pl.ds(k * _LANES, _LANES)]
            hv = half_v[j, pl.ds(k * _LANES, _LANES)]
            for b2 in range(_LANES):
                s = sv[b2]
                h = hv[b2]
                b = k * _LANES + b2
                for c in range(EMB_DIM // _LANES):
                    src = pl.ds(h + c * _LANES, _LANES)
                    dst = pl.ds(c * _LANES, _LANES)
                    ob_v[b, 0, dst] = tiles_v[b, s, src] * rep_v[b, 0, dst]
            return carry

        lax.fori_loop(0, _CHUNK // _LANES, mul_body, 0)

        pltpu.sync_copy(ob_v, out_hbm.at[pl.ds(rbase, _CHUNK)])


@jax.jit
def kernel(_next_types, representation, table):
    idx = _next_types.reshape(_NW * _NCHUNK, _CHUNK).astype(jnp.int32)
    table3 = table.reshape(_GROUPS, 8, 2 * EMB_DIM)

    mesh = plsc.VectorSubcoreMesh(core_axis_name="c", subcore_axis_name="s")
    out = pl.kernel(
        _emb_mul_kernel,
        out_type=jax.ShapeDtypeStruct((BATCH, 1, EMB_DIM), jnp.float32),
        mesh=mesh,
        compiler_params=pltpu.CompilerParams(use_tc_tiling_on_sc=True),
        scratch_types=[
            pltpu.VMEM((_NCHUNK, _CHUNK), jnp.int32),
            pltpu.VMEM((_NCHUNK, _CHUNK), jnp.int32),
            pltpu.VMEM((_NCHUNK, _CHUNK), jnp.int32),
            pltpu.VMEM((_NCHUNK, _CHUNK), jnp.int32),
            pltpu.VMEM((_CHUNK, 8, 2 * EMB_DIM), jnp.float32),
            pltpu.VMEM((_CHUNK, 1, EMB_DIM), jnp.float32),
            pltpu.VMEM((_CHUNK, 1, EMB_DIM), jnp.float32),
            pltpu.SemaphoreType.DMA,
            pltpu.SemaphoreType.DMA,
        ],
    )(idx, representation, table3)
    return out


# speedup vs baseline: 1.7699x; 1.7699x over previous
import jax
import jax.numpy as jnp
from jax import lax
from jax.experimental import pallas as pl
from jax.experimental.pallas import tpu as pltpu
from jax.experimental.pallas import tpu_sc as plsc

VOCAB = 1000000
EMB_DIM = 64
BATCH = 16384

_NC = 2
_NS = 16
_LANES = 16
_NW = _NC * _NS
_BPW = BATCH // _NW
_CHUNK = 256
_NCHUNK = _BPW // _CHUNK


def _emb_mul_kernel(idx_hbm, repr_hbm, table_hbm, out_hbm,
                    idx_v, rows_v, rep_v, gsem, rsem):
    wid = lax.axis_index("s") * _NC + lax.axis_index("c")
    base = wid * _BPW

    pltpu.sync_copy(idx_hbm.at[pl.ds(wid * (_BPW // 128), _BPW // 128)],
                    idx_v)

    for j in range(_NCHUNK):
        rbase = base + j * _CHUNK
        rep_cp = pltpu.async_copy(repr_hbm.at[pl.ds(rbase, _CHUNK)],
                                  rep_v, rsem)

        def fire_body(k, carry):
            jj = (j * _CHUNK + k * _LANES) // 128
            t = (k * _LANES) % 128
            v = idx_v[jj, pl.ds(t, _LANES)]
            for b2 in range(_LANES):
                row = v[b2]
                pltpu.async_copy(
                    table_hbm.at[pl.ds(row, 1)],
                    rows_v.at[pl.ds(k * _LANES + b2, 1)],
                    gsem)
            return carry

        lax.fori_loop(0, _CHUNK // _LANES, fire_body, 0)

        pltpu.make_async_copy(
            table_hbm.at[pl.ds(0, _CHUNK)], rows_v, gsem).wait()
        rep_cp.wait()

        def mul_body(b, carry):
            for c in range(EMB_DIM // _LANES):
                sl = pl.ds(c * _LANES, _LANES)
                rows_v[b, sl] = rows_v[b, sl] * rep_v[b, sl]
            return carry

        lax.fori_loop(0, _CHUNK, mul_body, 0, unroll=4)

        pltpu.sync_copy(rows_v, out_hbm.at[pl.ds(rbase, _CHUNK)])


@jax.jit
def kernel(_next_types, representation, table):
    idx = _next_types.reshape(BATCH // 128, 128).astype(jnp.int32)
    rep = representation.reshape(BATCH, EMB_DIM)

    mesh = plsc.VectorSubcoreMesh(core_axis_name="c", subcore_axis_name="s")
    out = pl.kernel(
        _emb_mul_kernel,
        out_type=jax.ShapeDtypeStruct((BATCH, EMB_DIM), jnp.float32),
        mesh=mesh,
        compiler_params=pltpu.CompilerParams(use_tc_tiling_on_sc=True),
        scratch_types=[
            pltpu.VMEM((_BPW // 128, 128), jnp.int32),
            pltpu.VMEM((_CHUNK, EMB_DIM), jnp.float32),
            pltpu.VMEM((_CHUNK, EMB_DIM), jnp.float32),
            pltpu.SemaphoreType.DMA,
            pltpu.SemaphoreType.DMA,
        ],
    )(idx, rep, table)
    return out.reshape(BATCH, 1, EMB_DIM)


# speedup vs baseline: 1.7778x; 1.0044x over previous
import jax
import jax.numpy as jnp
from jax import lax
from jax.experimental import pallas as pl
from jax.experimental.pallas import tpu as pltpu
from jax.experimental.pallas import tpu_sc as plsc

VOCAB = 1000000
EMB_DIM = 64
BATCH = 16384

_NC = 2
_NS = 16
_LANES = 16
_NW = _NC * _NS
_BPW = BATCH // _NW
_CHUNK = 256
_NCHUNK = _BPW // _CHUNK


def _emb_mul_kernel(idx_hbm, repr_hbm, table_hbm, out_hbm,
                    idx_v, rows_v, rep_v, gsem, rsem):
    wid = lax.axis_index("s") * _NC + lax.axis_index("c")
    base = wid * _BPW

    pltpu.sync_copy(idx_hbm.at[pl.ds(wid * (_BPW // 128), _BPW // 128)],
                    idx_v)

    for j in range(_NCHUNK):
        rbase = base + j * _CHUNK
        rep_cp = pltpu.async_copy(repr_hbm.at[pl.ds(rbase, _CHUNK)],
                                  rep_v, rsem)

        def fire_body(k, carry):
            jj = (j * _CHUNK + k * _LANES) // 128
            t = (k * _LANES) % 128
            v = idx_v[jj, pl.ds(t, _LANES)]
            for b2 in range(_LANES):
                row = v[b2]
                pltpu.async_copy(
                    table_hbm.at[pl.ds(row, 1)],
                    rows_v.at[pl.ds(k * _LANES + b2, 1)],
                    gsem)
            return carry

        with jax.named_scope("fire"):
            lax.fori_loop(0, _CHUNK // _LANES, fire_body, 0)

        with jax.named_scope("drain"):
            pltpu.make_async_copy(
                table_hbm.at[pl.ds(0, _CHUNK)], rows_v, gsem).wait()
            rep_cp.wait()

        def mul_body(b, carry):
            for c in range(EMB_DIM // _LANES):
                sl = pl.ds(c * _LANES, _LANES)
                rows_v[b, sl] = rows_v[b, sl] * rep_v[b, sl]
            return carry

        with jax.named_scope("mul"):
            lax.fori_loop(0, _CHUNK, mul_body, 0, unroll=4)

        with jax.named_scope("writeout"):
            pltpu.sync_copy(rows_v, out_hbm.at[pl.ds(rbase, _CHUNK)])


@jax.jit
def kernel(_next_types, representation, table):
    idx = _next_types.reshape(BATCH // 128, 128).astype(jnp.int32)
    rep = representation.reshape(BATCH, EMB_DIM)

    mesh = plsc.VectorSubcoreMesh(core_axis_name="c", subcore_axis_name="s")
    out = pl.kernel(
        _emb_mul_kernel,
        out_type=jax.ShapeDtypeStruct((BATCH, EMB_DIM), jnp.float32),
        mesh=mesh,
        compiler_params=pltpu.CompilerParams(use_tc_tiling_on_sc=True),
        scratch_types=[
            pltpu.VMEM((_BPW // 128, 128), jnp.int32),
            pltpu.VMEM((_CHUNK, EMB_DIM), jnp.float32),
            pltpu.VMEM((_CHUNK, EMB_DIM), jnp.float32),
            pltpu.SemaphoreType.DMA,
            pltpu.SemaphoreType.DMA,
        ],
    )(idx, rep, table)
    return out.reshape(BATCH, 1, EMB_DIM)
